# initial kernel scaffold (unmeasured)
import jax
import jax.numpy as jnp
from jax import lax
from jax.experimental import pallas as pl
from jax.experimental.pallas import tpu as pltpu

N_DEV = 32
M = 4096
N_COLS = 2048
CHUNK = M // N_DEV


def _all_reduce_body(p_ref, out_ref, comm_ref, send_sems, recv_sems, credit_sem):
    me = lax.axis_index("i")
    left = lax.rem(me + (N_DEV - 1), N_DEV)
    right = lax.rem(me + 1, N_DEV)

    barrier = pltpu.get_barrier_semaphore()
    for nbr in (left, right):
        pl.semaphore_signal(
            barrier, inc=1, device_id=(nbr,), device_id_type=pl.DeviceIdType.MESH
        )
    pl.semaphore_wait(barrier, 2)

    out_ref[...] = p_ref[...]

    n_steps = 2 * (N_DEV - 1)
    for t in range(n_steps):
        slot = t % 2
        is_rs = t < N_DEV - 1
        s = t if is_rs else t - (N_DEV - 1)
        if is_rs:
            send_idx = lax.rem(me - s + N_DEV, N_DEV)
            recv_idx = lax.rem(me - s - 1 + 2 * N_DEV, N_DEV)
        else:
            send_idx = lax.rem(me + 1 - s + N_DEV, N_DEV)
            recv_idx = lax.rem(me - s + N_DEV, N_DEV)

        if t >= 2:
            pl.semaphore_wait(credit_sem, 1)

        src = out_ref.at[pl.ds(send_idx * CHUNK, CHUNK)]
        if is_rs:
            dst = comm_ref.at[slot]
        else:
            dst = out_ref.at[pl.ds(send_idx * CHUNK, CHUNK)]
        rdma = pltpu.make_async_remote_copy(
            src_ref=src,
            dst_ref=dst,
            send_sem=send_sems.at[slot],
            recv_sem=recv_sems.at[slot],
            device_id=(right,),
            device_id_type=pl.DeviceIdType.MESH,
        )
        rdma.start()
        rdma.wait_send()
        rdma.wait_recv()

        if t < n_steps - 2:
            pl.semaphore_signal(
                credit_sem, inc=1, device_id=(left,),
                device_id_type=pl.DeviceIdType.MESH,
            )
        if is_rs:
            out_ref[pl.ds(recv_idx * CHUNK, CHUNK)] += comm_ref[slot]


def kernel(x, w_mat):
    partial = jnp.dot(x, w_mat, preferred_element_type=jnp.float32)
    return pl.pallas_call(
        _all_reduce_body,
        out_shape=jax.ShapeDtypeStruct((M, N_COLS), jnp.float32),
        in_specs=[pl.BlockSpec(memory_space=pltpu.VMEM)],
        out_specs=pl.BlockSpec(memory_space=pltpu.VMEM),
        scratch_shapes=[
            pltpu.VMEM((2, CHUNK, N_COLS), jnp.float32),
            pltpu.SemaphoreType.DMA((2,)),
            pltpu.SemaphoreType.DMA((2,)),
            pltpu.SemaphoreType.REGULAR,
        ],
        compiler_params=pltpu.CompilerParams(collective_id=0),
    )(partial)


# baseline (device time: 878312 ns/iter reference)
import jax
import jax.numpy as jnp
from jax import lax
from jax.experimental import pallas as pl
from jax.experimental.pallas import tpu as pltpu

N_DEV = 32
M = 4096
N_COLS = 2048
CHUNK = M // N_DEV


def _all_reduce_body(p_ref, out_ref, comm_ref, send_sems, recv_sems, credit_sem,
                     copy_sem):
    me = lax.axis_index("i")
    left = lax.rem(me + (N_DEV - 1), N_DEV)
    right = lax.rem(me + 1, N_DEV)

    hbm_copy = pltpu.make_async_copy(p_ref, out_ref, copy_sem)
    hbm_copy.start()

    barrier = pltpu.get_barrier_semaphore()
    for nbr in (left, right):
        pl.semaphore_signal(
            barrier, inc=1, device_id=(nbr,), device_id_type=pl.DeviceIdType.MESH
        )
    pl.semaphore_wait(barrier, 2)

    hbm_copy.wait()

    n_steps = 2 * (N_DEV - 1)
    for t in range(n_steps):
        slot = t % 2
        is_rs = t < N_DEV - 1
        s = t if is_rs else t - (N_DEV - 1)
        if is_rs:
            send_idx = lax.rem(me - s + N_DEV, N_DEV)
            recv_idx = lax.rem(me - s - 1 + 2 * N_DEV, N_DEV)
        else:
            send_idx = lax.rem(me + 1 - s + N_DEV, N_DEV)
            recv_idx = lax.rem(me - s + N_DEV, N_DEV)

        if t >= 2:
            pl.semaphore_wait(credit_sem, 1)

        src = out_ref.at[pl.ds(send_idx * CHUNK, CHUNK)]
        if is_rs:
            dst = comm_ref.at[slot]
        else:
            dst = out_ref.at[pl.ds(send_idx * CHUNK, CHUNK)]
        rdma = pltpu.make_async_remote_copy(
            src_ref=src,
            dst_ref=dst,
            send_sem=send_sems.at[slot],
            recv_sem=recv_sems.at[slot],
            device_id=(right,),
            device_id_type=pl.DeviceIdType.MESH,
        )
        rdma.start()
        rdma.wait_send()
        rdma.wait_recv()

        if t < n_steps - 2:
            pl.semaphore_signal(
                credit_sem, inc=1, device_id=(left,),
                device_id_type=pl.DeviceIdType.MESH,
            )
        if is_rs:
            out_ref[pl.ds(recv_idx * CHUNK, CHUNK)] += comm_ref[slot]


def kernel(x, w_mat):
    partial = jnp.dot(x, w_mat, preferred_element_type=jnp.float32)
    return pl.pallas_call(
        _all_reduce_body,
        out_shape=jax.ShapeDtypeStruct((M, N_COLS), jnp.float32),
        in_specs=[pl.BlockSpec(memory_space=pltpu.MemorySpace.HBM)],
        out_specs=pl.BlockSpec(memory_space=pltpu.VMEM),
        scratch_shapes=[
            pltpu.VMEM((2, CHUNK, N_COLS), jnp.float32),
            pltpu.SemaphoreType.DMA((2,)),
            pltpu.SemaphoreType.DMA((2,)),
            pltpu.SemaphoreType.REGULAR,
            pltpu.SemaphoreType.DMA,
        ],
        compiler_params=pltpu.CompilerParams(
            collective_id=0, vmem_limit_bytes=40 * 1024 * 1024
        ),
    )(partial)


# device time: 520632 ns/iter; 1.6870x vs baseline; 1.6870x over previous
import jax
import jax.numpy as jnp
from jax import lax
from jax.experimental import pallas as pl
from jax.experimental.pallas import tpu as pltpu

N_DEV = 32
M = 4096
N_COLS = 2048
HALF = N_COLS // 2
CHUNK = M // N_DEV


def _me_to_r(me):
    z = me // 8
    p = me % 8
    y = p // 2
    q = p % 2
    x = jnp.where(y % 2 == 0, q, 1 - q)
    k = 4 * z + jnp.where(z % 2 == 0, y, 3 - y)
    return jnp.where(x == 0, k, (N_DEV - 1) - k)


def _r_to_me(r):
    r = lax.rem(r + N_DEV, N_DEV)
    x = jnp.where(r < 16, 0, 1)
    k = jnp.where(r < 16, r, (N_DEV - 1) - r)
    z = k // 4
    yy = k % 4
    y = jnp.where(z % 2 == 0, yy, 3 - yy)
    p = 2 * y + jnp.where(y % 2 == 0, x, 1 - x)
    return 8 * z + p


def _all_reduce_body(
    p_ref, out_ref, commf_ref, commb_ref,
    sendf_sems, recvf_sems, sendb_sems, recvb_sems,
    creditf_sem, creditb_sem, copy_sem,
):
    me = lax.axis_index("i")
    r = _me_to_r(me)
    nxt = _r_to_me(r + 1)
    prv = _r_to_me(r - 1)

    hbm_copy = pltpu.make_async_copy(p_ref, out_ref, copy_sem)
    hbm_copy.start()

    barrier = pltpu.get_barrier_semaphore()
    for nbr in (prv, nxt):
        pl.semaphore_signal(
            barrier, inc=1, device_id=(nbr,), device_id_type=pl.DeviceIdType.MESH
        )
    pl.semaphore_wait(barrier, 2)

    hbm_copy.wait()

    n_steps = 2 * (N_DEV - 1)
    for t in range(n_steps):
        slot = t % 2
        is_rs = t < N_DEV - 1
        s = t if is_rs else t - (N_DEV - 1)
        if is_rs:
            f_send = lax.rem(r - s + N_DEV, N_DEV)
            f_recv = lax.rem(r - s - 1 + 2 * N_DEV, N_DEV)
            b_send = lax.rem(r + s, N_DEV)
            b_recv = lax.rem(r + s + 1, N_DEV)
        else:
            f_send = lax.rem(r + 1 - s + N_DEV, N_DEV)
            f_recv = lax.rem(r - s + N_DEV, N_DEV)
            b_send = lax.rem(r - 1 + s + N_DEV, N_DEV)
            b_recv = lax.rem(r + s, N_DEV)

        if t >= 2:
            pl.semaphore_wait(creditf_sem, 1)
            pl.semaphore_wait(creditb_sem, 1)

        srcf = out_ref.at[pl.ds(f_send * CHUNK, CHUNK), pl.ds(0, HALF)]
        srcb = out_ref.at[pl.ds(b_send * CHUNK, CHUNK), pl.ds(HALF, HALF)]
        if is_rs:
            dstf = commf_ref.at[slot]
            dstb = commb_ref.at[slot]
        else:
            dstf = out_ref.at[pl.ds(f_send * CHUNK, CHUNK), pl.ds(0, HALF)]
            dstb = out_ref.at[pl.ds(b_send * CHUNK, CHUNK), pl.ds(HALF, HALF)]
        rdmaf = pltpu.make_async_remote_copy(
            src_ref=srcf, dst_ref=dstf,
            send_sem=sendf_sems.at[slot], recv_sem=recvf_sems.at[slot],
            device_id=(nxt,), device_id_type=pl.DeviceIdType.MESH,
        )
        rdmab = pltpu.make_async_remote_copy(
            src_ref=srcb, dst_ref=dstb,
            send_sem=sendb_sems.at[slot], recv_sem=recvb_sems.at[slot],
            device_id=(prv,), device_id_type=pl.DeviceIdType.MESH,
        )
        rdmaf.start()
        rdmab.start()
        rdmaf.wait_send()
        rdmab.wait_send()
        rdmaf.wait_recv()
        rdmab.wait_recv()

        if is_rs:
            out_ref[pl.ds(f_recv * CHUNK, CHUNK), pl.ds(0, HALF)] += commf_ref[slot]
            out_ref[pl.ds(b_recv * CHUNK, CHUNK), pl.ds(HALF, HALF)] += commb_ref[slot]

        if t < n_steps - 2:
            pl.semaphore_signal(
                creditf_sem, inc=1, device_id=(prv,),
                device_id_type=pl.DeviceIdType.MESH,
            )
            pl.semaphore_signal(
                creditb_sem, inc=1, device_id=(nxt,),
                device_id_type=pl.DeviceIdType.MESH,
            )


def kernel(x, w_mat):
    partial = jnp.dot(x, w_mat, preferred_element_type=jnp.float32)
    return pl.pallas_call(
        _all_reduce_body,
        out_shape=jax.ShapeDtypeStruct((M, N_COLS), jnp.float32),
        in_specs=[pl.BlockSpec(memory_space=pltpu.MemorySpace.HBM)],
        out_specs=pl.BlockSpec(memory_space=pltpu.VMEM),
        scratch_shapes=[
            pltpu.VMEM((2, CHUNK, HALF), jnp.float32),
            pltpu.VMEM((2, CHUNK, HALF), jnp.float32),
            pltpu.SemaphoreType.DMA((2,)),
            pltpu.SemaphoreType.DMA((2,)),
            pltpu.SemaphoreType.DMA((2,)),
            pltpu.SemaphoreType.DMA((2,)),
            pltpu.SemaphoreType.REGULAR,
            pltpu.SemaphoreType.REGULAR,
            pltpu.SemaphoreType.DMA,
        ],
        compiler_params=pltpu.CompilerParams(
            collective_id=0, vmem_limit_bytes=40 * 1024 * 1024
        ),
    )(partial)


# device time: 413455 ns/iter; 2.1243x vs baseline; 1.2592x over previous
import jax
import jax.numpy as jnp
from jax import lax
from jax.experimental import pallas as pl
from jax.experimental.pallas import tpu as pltpu

N_DEV = 32
M = 4096
N_COLS = 2048
HALF = N_COLS // 2
CHUNK = M // N_DEV
NSUB = 2
W = HALF // NSUB
NRINGS = 2 * NSUB
_ORDER = [0, NSUB, 1, NSUB + 1] if NSUB == 2 else list(range(NRINGS))
N_STEPS = 2 * (N_DEV - 1)


def _me_to_r(me):
    z = me // 8
    p = me % 8
    y = p // 2
    q = p % 2
    x = jnp.where(y % 2 == 0, q, 1 - q)
    k = 4 * z + jnp.where(z % 2 == 0, y, 3 - y)
    return jnp.where(x == 0, k, (N_DEV - 1) - k)


def _r_to_me(r):
    r = lax.rem(r + N_DEV, N_DEV)
    x = jnp.where(r < 16, 0, 1)
    k = jnp.where(r < 16, r, (N_DEV - 1) - r)
    z = k // 4
    yy = k % 4
    y = jnp.where(z % 2 == 0, yy, 3 - yy)
    p = 2 * y + jnp.where(y % 2 == 0, x, 1 - x)
    return 8 * z + p


def _ring_idx(r, t, forward):
    is_rs = t < N_DEV - 1
    s = t if is_rs else t - (N_DEV - 1)
    if forward:
        if is_rs:
            send = lax.rem(r - s + N_DEV, N_DEV)
            recv = lax.rem(r - s - 1 + 2 * N_DEV, N_DEV)
        else:
            send = lax.rem(r + 1 - s + N_DEV, N_DEV)
            recv = lax.rem(r - s + N_DEV, N_DEV)
    else:
        if is_rs:
            send = lax.rem(r + s, N_DEV)
            recv = lax.rem(r + s + 1, N_DEV)
        else:
            send = lax.rem(r - 1 + s + N_DEV, N_DEV)
            recv = lax.rem(r + s, N_DEV)
    return send, recv


def _all_reduce_body(
    p_ref, out_ref, comm_ref, send_sems, recv_sems, credit_sems, copy_sem
):
    me = lax.axis_index("i")
    r = _me_to_r(me)
    nxt = _r_to_me(r + 1)
    prv = _r_to_me(r - 1)

    hbm_copy = pltpu.make_async_copy(p_ref, out_ref, copy_sem)
    hbm_copy.start()

    barrier = pltpu.get_barrier_semaphore()
    for nbr in (prv, nxt):
        pl.semaphore_signal(
            barrier, inc=1, device_id=(nbr,), device_id_type=pl.DeviceIdType.MESH
        )
    pl.semaphore_wait(barrier, 2)

    hbm_copy.wait()

    def start(q, t):
        forward = q < NSUB
        slot = t % 2
        send_idx, _ = _ring_idx(r, t, forward)
        col0 = q * W
        src = out_ref.at[pl.ds(send_idx * CHUNK, CHUNK), pl.ds(col0, W)]
        if t < N_DEV - 1:
            dst = comm_ref.at[q, slot]
        else:
            dst = out_ref.at[pl.ds(send_idx * CHUNK, CHUNK), pl.ds(col0, W)]
        rdma = pltpu.make_async_remote_copy(
            src_ref=src,
            dst_ref=dst,
            send_sem=send_sems.at[q, slot],
            recv_sem=recv_sems.at[q, slot],
            device_id=(nxt if forward else prv,),
            device_id_type=pl.DeviceIdType.MESH,
        )
        rdma.start()
        return rdma

    def credit_wait(q, t):
        if t >= 2:
            pl.semaphore_wait(credit_sems.at[q], 1)

    def finalize(q, t, rdma):
        forward = q < NSUB
        slot = t % 2
        _, recv_idx = _ring_idx(r, t, forward)
        rdma.wait_send()
        rdma.wait_recv()
        if t < N_DEV - 1:
            col0 = q * W
            out_ref[pl.ds(recv_idx * CHUNK, CHUNK), pl.ds(col0, W)] += comm_ref[
                q, slot
            ]
        if t < N_STEPS - 2:
            pl.semaphore_signal(
                credit_sems.at[q],
                inc=1,
                device_id=(prv if forward else nxt,),
                device_id_type=pl.DeviceIdType.MESH,
            )

    pending = {}
    for q in _ORDER:
        credit_wait(q, 0)
        pending[q] = start(q, 0)
    for t in range(1, N_STEPS):
        for q in _ORDER:
            finalize(q, t - 1, pending[q])
            credit_wait(q, t)
            pending[q] = start(q, t)
    for q in _ORDER:
        finalize(q, N_STEPS - 1, pending[q])


def kernel(x, w_mat):
    partial = jnp.dot(x, w_mat, preferred_element_type=jnp.float32)
    return pl.pallas_call(
        _all_reduce_body,
        out_shape=jax.ShapeDtypeStruct((M, N_COLS), jnp.float32),
        in_specs=[pl.BlockSpec(memory_space=pltpu.MemorySpace.HBM)],
        out_specs=pl.BlockSpec(memory_space=pltpu.VMEM),
        scratch_shapes=[
            pltpu.VMEM((NRINGS, 2, CHUNK, W), jnp.float32),
            pltpu.SemaphoreType.DMA((NRINGS, 2)),
            pltpu.SemaphoreType.DMA((NRINGS, 2)),
            pltpu.SemaphoreType.REGULAR((NRINGS,)),
            pltpu.SemaphoreType.DMA,
        ],
        compiler_params=pltpu.CompilerParams(
            collective_id=0, vmem_limit_bytes=40 * 1024 * 1024
        ),
    )(partial)
